# baseline (speedup 1.0000x reference)
import jax
import jax.numpy as jnp
from jax.experimental import pallas as pl
from jax.experimental.pallas import tpu as pltpu

CHUNK = 128
TAU = 0.7


def _mlp_chunk_logits_kernel(x_ref, w1_ref, b1_ref, w2_ref, b2_ref, out_ref):
    x = x_ref[...]
    h = jax.lax.dot_general(
        x, w1_ref[...], (((1,), (1,)), ((), ())),
        preferred_element_type=jnp.float32)
    h = jnp.maximum(h + b1_ref[...], 0.0)
    T, H = h.shape
    nc = T // CHUNK
    hm = jnp.mean(h.reshape(nc, CHUNK, H), axis=1)
    cl = jax.lax.dot_general(
        hm, w2_ref[...], (((1,), (1,)), ((), ())),
        preferred_element_type=jnp.float32)
    out_ref[0] = cl + b2_ref[...]


def _sticky_route_kernel(cl_ref, rw_ref, idx_ref):
    cl = cl_ref[0]
    C, E = cl.shape
    iota_e1 = jax.lax.broadcasted_iota(jnp.int32, (1, E), 1)
    iota_c1 = jax.lax.broadcasted_iota(jnp.int32, (1, C), 1)
    iota_c2 = jax.lax.broadcasted_iota(jnp.int32, (C, E), 0)
    iota_e2 = jax.lax.broadcasted_iota(jnp.int32, (C, E), 1)
    idx_vec = jnp.zeros((1, C), jnp.int32)
    oh = jnp.zeros((C, E), jnp.float32)
    prev = jnp.int32(0)
    for c in range(C):
        row = cl[c:c + 1, :]
        m = jnp.max(row)
        cand = jnp.min(jnp.where(row == m, iota_e1, E)).astype(jnp.int32)
        if c == 0:
            cur = cand
        else:
            prev_logit = jnp.sum(jnp.where(iota_e1 == prev, row, 0.0))
            cur = jnp.where((m - prev_logit) > TAU, cand, prev)
        idx_vec = jnp.where(iota_c1 == c, cur, idx_vec)
        oh = jnp.where((iota_c2 == c) & (iota_e2 == cur), 1.0, oh)
        prev = cur
    idx_ref[0] = idx_vec
    rw = jnp.broadcast_to(oh[:, None, :], (C, CHUNK, E)).reshape(C * CHUNK, E)
    rw_ref[0] = rw


def kernel(x, W1, b1, W2, b2):
    B, S, D = x.shape
    H = W1.shape[0]
    E = W2.shape[0]
    C = S // CHUNK
    TOK = 512
    nsteps = (B * S) // TOK
    x2 = x.reshape(B * S, D)

    cl = pl.pallas_call(
        _mlp_chunk_logits_kernel,
        grid=(nsteps,),
        in_specs=[
            pl.BlockSpec((TOK, D), lambda i: (i, 0)),
            pl.BlockSpec((H, D), lambda i: (0, 0)),
            pl.BlockSpec((1, H), lambda i: (0, 0)),
            pl.BlockSpec((E, H), lambda i: (0, 0)),
            pl.BlockSpec((1, E), lambda i: (0, 0)),
        ],
        out_specs=pl.BlockSpec((1, TOK // CHUNK, E), lambda i: (i, 0, 0)),
        out_shape=jax.ShapeDtypeStruct((nsteps, TOK // CHUNK, E), jnp.float32),
        compiler_params=pltpu.CompilerParams(
            dimension_semantics=("parallel",)),
    )(x2, W1, b1.reshape(1, H), W2, b2.reshape(1, E))

    cl3 = cl.reshape(B, C, E)

    rw, idx3 = pl.pallas_call(
        _sticky_route_kernel,
        grid=(B,),
        in_specs=[pl.BlockSpec((1, C, E), lambda b: (b, 0, 0))],
        out_specs=[
            pl.BlockSpec((1, S, E), lambda b: (b, 0, 0)),
            pl.BlockSpec((1, 1, C), lambda b: (b, 0, 0)),
        ],
        out_shape=[
            jax.ShapeDtypeStruct((B, S, E), jnp.float32),
            jax.ShapeDtypeStruct((B, 1, C), jnp.int32),
        ],
    )(cl3)

    return rw, idx3.reshape(B, C)
